# baseline (speedup 1.0000x reference)
import functools

import jax
import jax.numpy as jnp
from jax import lax
from jax.experimental import pallas as pl
from jax.experimental.pallas import tpu as pltpu
from jax.experimental.pallas import tpu_sc as plsc

N_ITEM = 6000
N_USER = 4000
N = N_ITEM + N_USER
E = 320000
D = 128

NC = 2
NS = 16
NW = NC * NS

RB = 400
NP = 10112
NBLK = -(-NP // RB)
ITEM_BLKS = N_ITEM // RB
HR = NP // 128

K = 128
EPT = 10368
E_PAD = NW * EPT
CHUNKS = EPT // K
NSLOT = 3
RPT = NP // NS

_mesh = plsc.VectorSubcoreMesh(core_axis_name="c", subcore_axis_name="s")


@functools.partial(
    pl.kernel,
    out_type=jax.ShapeDtypeStruct((2, NW, HR, 128), jnp.float32),
    mesh=_mesh,
    scratch_types=[
        pltpu.VMEM((1024,), jnp.int32),
        pltpu.VMEM((HR, 128), jnp.float32),
        pltpu.VMEM((HR, 128), jnp.float32),
    ],
    compiler_params=pltpu.CompilerParams(needs_layout_passes=False),
)
def _degrees_sc(src_hbm, dst_hbm, out_hbm, ebuf, hs, hd):
    c = lax.axis_index("c")
    s = lax.axis_index("s")
    wid = c * NS + s
    base = wid * EPT

    zero16 = jnp.zeros((16,), jnp.float32)
    one16 = jnp.ones((16,), jnp.float32)

    def zbody(j, _):
        hs[j // 8, pl.ds((j % 8) * 16, 16)] = zero16
        hd[j // 8, pl.ds((j % 8) * 16, 16)] = zero16
        return _

    lax.fori_loop(0, HR * 8, zbody, 0)

    def run(idx_hbm, hist):
        def cbody(ci, _):
            pltpu.sync_copy(idx_hbm.at[pl.ds(base + ci * 1024, 1024)], ebuf)

            def ibody(j, _):
                idx = ebuf[pl.ds(j * 16, 16)]
                plsc.addupdate_scatter(
                    hist,
                    [lax.shift_right_logical(idx, 7),
                     lax.bitwise_and(idx, 127)],
                    one16,
                )
                return _

            lax.fori_loop(0, 64, ibody, 0)
            return _

        lax.fori_loop(0, EPT // 1024, cbody, 0)

    run(src_hbm, hs)
    run(dst_hbm, hd)
    pltpu.sync_copy(hs, out_hbm.at[0, wid])
    pltpu.sync_copy(hd, out_hbm.at[1, wid])


@functools.partial(
    pl.kernel,
    out_type=jax.ShapeDtypeStruct((NC, NP, D), jnp.float32),
    mesh=_mesh,
    scratch_types=(
        [pltpu.VMEM((K,), jnp.int32) for _ in range(2 * NSLOT)]
        + [pltpu.VMEM((K, D), jnp.float32) for _ in range(NSLOT)]
        + [pltpu.VMEM_SHARED((NP, D), jnp.float32)]
        + [pltpu.SemaphoreType.DMA for _ in range(NSLOT)]
    ),
    compiler_params=pltpu.CompilerParams(needs_layout_passes=False),
)
def _edge_agg_sc(h_hbm, src_hbm, dst_hbm, zeros_hbm, out_hbm, *sc):
    sidx = sc[0:NSLOT]
    didx = sc[NSLOT:2 * NSLOT]
    buf = sc[2 * NSLOT:3 * NSLOT]
    accum = sc[3 * NSLOT]
    sem = sc[3 * NSLOT + 1:]
    c = lax.axis_index("c")
    s = lax.axis_index("s")
    wid = c * NS + s
    r0 = s * RPT

    pltpu.sync_copy(zeros_hbm, accum.at[pl.ds(r0, RPT)])
    plsc.subcore_barrier()

    for b in range(NSLOT):
        pltpu.sync_copy(src_hbm.at[wid, b], sidx[b])
        pltpu.sync_copy(dst_hbm.at[wid, b], didx[b])
        pltpu.async_copy(h_hbm.at[sidx[b]], buf[b], sem[b])

    def slot(ci, b):
        pltpu.make_async_copy(h_hbm.at[sidx[b]], buf[b], sem[b]).wait()
        pltpu.sync_copy(buf[b], accum.at[didx[b]], add=True)

        @pl.when(ci + NSLOT < CHUNKS)
        def _start_next():
            pltpu.sync_copy(src_hbm.at[wid, ci + NSLOT], sidx[b])
            pltpu.sync_copy(dst_hbm.at[wid, ci + NSLOT], didx[b])
            pltpu.async_copy(h_hbm.at[sidx[b]], buf[b], sem[b])

    def cbody(i, _):
        c0 = i * NSLOT
        for b in range(NSLOT):
            slot(c0 + b, b)
        return _

    lax.fori_loop(0, CHUNKS // NSLOT, cbody, 0)
    plsc.subcore_barrier()
    pltpu.sync_copy(accum.at[pl.ds(r0, RPT)], out_hbm.at[c, pl.ds(r0, RPT)])


def _proj_body(x_ref, w_ref, b_ref, o_ref):
    o_ref[...] = (
        jnp.dot(x_ref[...], w_ref[0], preferred_element_type=jnp.float32)
        + b_ref[0, 0]
    )


def _proj(x_pad, w_stack, b_stack):
    return pl.pallas_call(
        _proj_body,
        grid=(NBLK,),
        in_specs=[
            pl.BlockSpec((RB, D), lambda i: (i, 0)),
            pl.BlockSpec((1, D, D), lambda i: (i // ITEM_BLKS, 0, 0)),
            pl.BlockSpec((1, 1, D), lambda i: (i // ITEM_BLKS, 0, 0)),
        ],
        out_specs=pl.BlockSpec((RB, D), lambda i: (i, 0)),
        out_shape=jax.ShapeDtypeStruct((NP, D), jnp.float32),
    )(x_pad, w_stack, b_stack)


def _norms_body(degp_ref, o_ref):
    deg = jnp.sum(degp_ref[...], axis=1)
    o_ref[...] = lax.rsqrt(jnp.maximum(deg, 1.0)).T


def _norms(deg_partials):
    return pl.pallas_call(
        _norms_body,
        out_shape=jax.ShapeDtypeStruct((NP, 2), jnp.float32),
    )(deg_partials)


def _conv_in_body(x_ref, w_ref, nrm_ref, o_ref):
    h = jnp.dot(x_ref[...], w_ref[...], preferred_element_type=jnp.float32)
    o_ref[...] = h * nrm_ref[:, 0:1]


def _conv_in(x, w, norms):
    return pl.pallas_call(
        _conv_in_body,
        grid=(NBLK,),
        in_specs=[
            pl.BlockSpec((RB, D), lambda i: (i, 0)),
            pl.BlockSpec((D, D), lambda i: (0, 0)),
            pl.BlockSpec((RB, 2), lambda i: (i, 0)),
        ],
        out_specs=pl.BlockSpec((RB, D), lambda i: (i, 0)),
        out_shape=jax.ShapeDtypeStruct((NP, D), jnp.float32),
    )(x, w, norms)


def _conv_mid_body(agg_ref, nrm_ref, b_ref, w_ref, o_ref):
    t = (agg_ref[0] + agg_ref[1]) * nrm_ref[:, 1:2] + b_ref[...]
    h = jnp.dot(t, w_ref[...], preferred_element_type=jnp.float32)
    o_ref[...] = h * nrm_ref[:, 0:1]


def _conv_mid(agg, norms, b_prev, w_next):
    return pl.pallas_call(
        _conv_mid_body,
        grid=(NBLK,),
        in_specs=[
            pl.BlockSpec((NC, RB, D), lambda i: (0, i, 0)),
            pl.BlockSpec((RB, 2), lambda i: (i, 0)),
            pl.BlockSpec((1, D), lambda i: (0, 0)),
            pl.BlockSpec((D, D), lambda i: (0, 0)),
        ],
        out_specs=pl.BlockSpec((RB, D), lambda i: (i, 0)),
        out_shape=jax.ShapeDtypeStruct((NP, D), jnp.float32),
    )(agg, norms, b_prev, w_next)


def _dec_body(agg_ref, nrm_ref, bc_ref, w1_ref, b1_ref, w2_ref, b2_ref,
              w3_ref, b3_ref, wo_ref, bo_ref, o_ref):
    t = (agg_ref[0] + agg_ref[1]) * nrm_ref[:, 1:2] + bc_ref[...]
    d = jax.nn.relu(jnp.dot(t, w1_ref[...],
                            preferred_element_type=jnp.float32) + b1_ref[...])
    d = jax.nn.relu(jnp.dot(d, w2_ref[...],
                            preferred_element_type=jnp.float32) + b2_ref[...])
    d = jax.nn.relu(jnp.dot(d, w3_ref[...],
                            preferred_element_type=jnp.float32) + b3_ref[...])
    o = jnp.dot(d, wo_ref[0], preferred_element_type=jnp.float32) + bo_ref[0, 0]
    o_ref[...] = jax.nn.sigmoid(o)


def _decoder(agg, norms, conv_b, w1, b1, w2, b2, w3, b3, wo_stack, bo_stack):
    return pl.pallas_call(
        _dec_body,
        grid=(NBLK,),
        in_specs=[
            pl.BlockSpec((NC, RB, D), lambda i: (0, i, 0)),
            pl.BlockSpec((RB, 2), lambda i: (i, 0)),
            pl.BlockSpec((1, D), lambda i: (0, 0)),
            pl.BlockSpec((D, 3 * D), lambda i: (0, 0)),
            pl.BlockSpec((1, 3 * D), lambda i: (0, 0)),
            pl.BlockSpec((3 * D, 2 * D), lambda i: (0, 0)),
            pl.BlockSpec((1, 2 * D), lambda i: (0, 0)),
            pl.BlockSpec((2 * D, D), lambda i: (0, 0)),
            pl.BlockSpec((1, D), lambda i: (0, 0)),
            pl.BlockSpec((1, D, D), lambda i: (i // ITEM_BLKS, 0, 0)),
            pl.BlockSpec((1, 1, D), lambda i: (i // ITEM_BLKS, 0, 0)),
        ],
        out_specs=pl.BlockSpec((RB, D), lambda i: (i, 0)),
        out_shape=jax.ShapeDtypeStruct((NP, D), jnp.float32),
    )(agg, norms, conv_b, w1, b1, w2, b2, w3, b3, wo_stack, bo_stack)


def kernel(x_item, x_user, edge_index,
           W_in_item, b_in_item, W_in_user, b_in_user,
           conv0_W, conv0_b, conv1_W, conv1_b,
           mlp_W1, mlp_b1, mlp_W2, mlp_b2, mlp_W3, mlp_b3,
           W_out_item, b_out_item, W_out_user, b_out_user):
    f32 = jnp.float32

    x_cat = jnp.concatenate([x_item, x_user], axis=0)
    x_pad = jnp.pad(x_cat, ((0, NP - N), (0, 0)))

    pad_idx = jnp.full((E_PAD - E,), N, dtype=jnp.int32)
    src = jnp.concatenate([edge_index[0], pad_idx])
    dst = jnp.concatenate([edge_index[1], pad_idx])
    src3 = src.reshape(NW, CHUNKS, K)
    dst3 = dst.reshape(NW, CHUNKS, K)

    w_in = jnp.stack([W_in_item, W_in_user])
    b_in = jnp.stack([b_in_item, b_in_user]).reshape(2, 1, D)
    w_out = jnp.stack([W_out_item, W_out_user])
    b_out = jnp.stack([b_out_item, b_out_user]).reshape(2, 1, D)
    zeros_tile = jnp.zeros((RPT, D), f32)

    deg_partials = _degrees_sc(src, dst)
    norms = _norms(deg_partials.reshape(2, NW, NP))

    x_proj = _proj(x_pad, w_in, b_in)

    h0 = _conv_in(x_proj, conv0_W, norms)
    agg0 = _edge_agg_sc(h0, src3, dst3, zeros_tile)

    h1 = _conv_mid(agg0, norms, conv0_b.reshape(1, D), conv1_W)
    agg1 = _edge_agg_sc(h1, src3, dst3, zeros_tile)

    out = _decoder(agg1, norms, conv1_b.reshape(1, D),
                   mlp_W1, mlp_b1.reshape(1, 3 * D),
                   mlp_W2, mlp_b2.reshape(1, 2 * D),
                   mlp_W3, mlp_b3.reshape(1, D),
                   w_out, b_out)

    return (out[:N_ITEM], out[N_ITEM:N])


# speedup vs baseline: 1.5939x; 1.5939x over previous
import functools

import jax
import jax.numpy as jnp
from jax import lax
from jax.experimental import pallas as pl
from jax.experimental.pallas import tpu as pltpu
from jax.experimental.pallas import tpu_sc as plsc

N_ITEM = 6000
N_USER = 4000
N = N_ITEM + N_USER
E = 320000
D = 128

NC = 2
NS = 16
NW = NC * NS

RB = 400
NP = 10112
NBLK = -(-NP // RB)
ITEM_BLKS = N_ITEM // RB
HR = NP // 128

K = 128
EPT = 10240
E_PAD = NW * EPT
CHUNKS = EPT // K
NSLOT = 2
RPT = NP // NS

_mesh = plsc.VectorSubcoreMesh(core_axis_name="c", subcore_axis_name="s")


@functools.partial(
    pl.kernel,
    out_type=jax.ShapeDtypeStruct((2, NW, HR, 128), jnp.float32),
    mesh=_mesh,
    scratch_types=[
        pltpu.VMEM((1024,), jnp.int32),
        pltpu.VMEM((HR, 128), jnp.float32),
        pltpu.VMEM((HR, 128), jnp.float32),
    ],
    compiler_params=pltpu.CompilerParams(needs_layout_passes=False),
)
def _degrees_sc(src_hbm, dst_hbm, out_hbm, ebuf, hs, hd):
    c = lax.axis_index("c")
    s = lax.axis_index("s")
    wid = c * NS + s
    base = wid * EPT

    zero16 = jnp.zeros((16,), jnp.float32)
    one16 = jnp.ones((16,), jnp.float32)

    def zbody(j, _):
        hs[j // 8, pl.ds((j % 8) * 16, 16)] = zero16
        hd[j // 8, pl.ds((j % 8) * 16, 16)] = zero16
        return _

    lax.fori_loop(0, HR * 8, zbody, 0)

    def run(idx_hbm, hist):
        def cbody(ci, _):
            pltpu.sync_copy(idx_hbm.at[pl.ds(base + ci * 1024, 1024)], ebuf)

            def ibody(j, _):
                idx = ebuf[pl.ds(j * 16, 16)]
                plsc.addupdate_scatter(
                    hist,
                    [lax.shift_right_logical(idx, 7),
                     lax.bitwise_and(idx, 127)],
                    one16,
                )
                return _

            lax.fori_loop(0, 64, ibody, 0)
            return _

        lax.fori_loop(0, EPT // 1024, cbody, 0)

    run(src_hbm, hs)
    run(dst_hbm, hd)
    pltpu.sync_copy(hs, out_hbm.at[0, wid])
    pltpu.sync_copy(hd, out_hbm.at[1, wid])


@functools.partial(
    pl.kernel,
    out_type=jax.ShapeDtypeStruct((NC, NP, D), jnp.float32),
    mesh=_mesh,
    scratch_types=(
        [pltpu.VMEM((K,), jnp.int32) for _ in range(2 * NSLOT)]
        + [pltpu.VMEM((K, D), jnp.float32) for _ in range(NSLOT)]
        + [pltpu.VMEM_SHARED((NP, D), jnp.float32)]
        + [pltpu.SemaphoreType.DMA for _ in range(NSLOT)]
    ),
    compiler_params=pltpu.CompilerParams(needs_layout_passes=False),
)
def _edge_agg_sc(h_hbm, src_hbm, dst_hbm, zeros_hbm, out_hbm, *sc):
    sidx = sc[0:NSLOT]
    didx = sc[NSLOT:2 * NSLOT]
    buf = sc[2 * NSLOT:3 * NSLOT]
    accum = sc[3 * NSLOT]
    sem = sc[3 * NSLOT + 1:]
    c = lax.axis_index("c")
    s = lax.axis_index("s")
    wid = c * NS + s
    r0 = s * RPT

    pltpu.sync_copy(zeros_hbm, accum.at[pl.ds(r0, RPT)])
    plsc.subcore_barrier()

    for b in range(NSLOT):
        pltpu.sync_copy(src_hbm.at[wid, b], sidx[b])
        pltpu.sync_copy(dst_hbm.at[wid, b], didx[b])
        pltpu.async_copy(h_hbm.at[sidx[b]], buf[b], sem[b])

    def slot(ci, b):
        pltpu.make_async_copy(h_hbm.at[sidx[b]], buf[b], sem[b]).wait()
        pltpu.sync_copy(buf[b], accum.at[didx[b]], add=True)

        @pl.when(ci + NSLOT < CHUNKS)
        def _start_next():
            pltpu.sync_copy(src_hbm.at[wid, ci + NSLOT], sidx[b])
            pltpu.sync_copy(dst_hbm.at[wid, ci + NSLOT], didx[b])
            pltpu.async_copy(h_hbm.at[sidx[b]], buf[b], sem[b])

    def cbody(i, _):
        c0 = i * NSLOT
        for b in range(NSLOT):
            slot(c0 + b, b)
        return _

    lax.fori_loop(0, CHUNKS // NSLOT, cbody, 0)
    plsc.subcore_barrier()
    pltpu.sync_copy(accum.at[pl.ds(r0, RPT)], out_hbm.at[c, pl.ds(r0, RPT)])


def _proj_body(x_ref, w_ref, b_ref, o_ref):
    o_ref[...] = (
        jnp.dot(x_ref[...], w_ref[0], preferred_element_type=jnp.float32)
        + b_ref[0, 0]
    )


def _proj(x_pad, w_stack, b_stack):
    return pl.pallas_call(
        _proj_body,
        grid=(NBLK,),
        in_specs=[
            pl.BlockSpec((RB, D), lambda i: (i, 0)),
            pl.BlockSpec((1, D, D), lambda i: (i // ITEM_BLKS, 0, 0)),
            pl.BlockSpec((1, 1, D), lambda i: (i // ITEM_BLKS, 0, 0)),
        ],
        out_specs=pl.BlockSpec((RB, D), lambda i: (i, 0)),
        out_shape=jax.ShapeDtypeStruct((NP, D), jnp.float32),
    )(x_pad, w_stack, b_stack)


def _norms_body(degp_ref, o_ref):
    deg = jnp.sum(degp_ref[...], axis=1)
    o_ref[...] = lax.rsqrt(jnp.maximum(deg, 1.0)).T


def _norms(deg_partials):
    return pl.pallas_call(
        _norms_body,
        out_shape=jax.ShapeDtypeStruct((NP, 2), jnp.float32),
    )(deg_partials)


def _conv_in_body(x_ref, w_ref, nrm_ref, o_ref):
    h = jnp.dot(x_ref[...], w_ref[...], preferred_element_type=jnp.float32)
    o_ref[...] = h * nrm_ref[:, 0:1]


def _conv_in(x, w, norms):
    return pl.pallas_call(
        _conv_in_body,
        grid=(NBLK,),
        in_specs=[
            pl.BlockSpec((RB, D), lambda i: (i, 0)),
            pl.BlockSpec((D, D), lambda i: (0, 0)),
            pl.BlockSpec((RB, 2), lambda i: (i, 0)),
        ],
        out_specs=pl.BlockSpec((RB, D), lambda i: (i, 0)),
        out_shape=jax.ShapeDtypeStruct((NP, D), jnp.float32),
    )(x, w, norms)


def _conv_mid_body(agg_ref, nrm_ref, b_ref, w_ref, o_ref):
    t = (agg_ref[0] + agg_ref[1]) * nrm_ref[:, 1:2] + b_ref[...]
    h = jnp.dot(t, w_ref[...], preferred_element_type=jnp.float32)
    o_ref[...] = h * nrm_ref[:, 0:1]


def _conv_mid(agg, norms, b_prev, w_next):
    return pl.pallas_call(
        _conv_mid_body,
        grid=(NBLK,),
        in_specs=[
            pl.BlockSpec((NC, RB, D), lambda i: (0, i, 0)),
            pl.BlockSpec((RB, 2), lambda i: (i, 0)),
            pl.BlockSpec((1, D), lambda i: (0, 0)),
            pl.BlockSpec((D, D), lambda i: (0, 0)),
        ],
        out_specs=pl.BlockSpec((RB, D), lambda i: (i, 0)),
        out_shape=jax.ShapeDtypeStruct((NP, D), jnp.float32),
    )(agg, norms, b_prev, w_next)


def _dec_body(agg_ref, nrm_ref, bc_ref, w1_ref, b1_ref, w2_ref, b2_ref,
              w3_ref, b3_ref, wo_ref, bo_ref, o_ref):
    t = (agg_ref[0] + agg_ref[1]) * nrm_ref[:, 1:2] + bc_ref[...]
    d = jax.nn.relu(jnp.dot(t, w1_ref[...],
                            preferred_element_type=jnp.float32) + b1_ref[...])
    d = jax.nn.relu(jnp.dot(d, w2_ref[...],
                            preferred_element_type=jnp.float32) + b2_ref[...])
    d = jax.nn.relu(jnp.dot(d, w3_ref[...],
                            preferred_element_type=jnp.float32) + b3_ref[...])
    o = jnp.dot(d, wo_ref[0], preferred_element_type=jnp.float32) + bo_ref[0, 0]
    o_ref[...] = jax.nn.sigmoid(o)


def _decoder(agg, norms, conv_b, w1, b1, w2, b2, w3, b3, wo_stack, bo_stack):
    return pl.pallas_call(
        _dec_body,
        grid=(NBLK,),
        in_specs=[
            pl.BlockSpec((NC, RB, D), lambda i: (0, i, 0)),
            pl.BlockSpec((RB, 2), lambda i: (i, 0)),
            pl.BlockSpec((1, D), lambda i: (0, 0)),
            pl.BlockSpec((D, 3 * D), lambda i: (0, 0)),
            pl.BlockSpec((1, 3 * D), lambda i: (0, 0)),
            pl.BlockSpec((3 * D, 2 * D), lambda i: (0, 0)),
            pl.BlockSpec((1, 2 * D), lambda i: (0, 0)),
            pl.BlockSpec((2 * D, D), lambda i: (0, 0)),
            pl.BlockSpec((1, D), lambda i: (0, 0)),
            pl.BlockSpec((1, D, D), lambda i: (i // ITEM_BLKS, 0, 0)),
            pl.BlockSpec((1, 1, D), lambda i: (i // ITEM_BLKS, 0, 0)),
        ],
        out_specs=pl.BlockSpec((RB, D), lambda i: (i, 0)),
        out_shape=jax.ShapeDtypeStruct((NP, D), jnp.float32),
    )(agg, norms, conv_b, w1, b1, w2, b2, w3, b3, wo_stack, bo_stack)


def kernel(x_item, x_user, edge_index,
           W_in_item, b_in_item, W_in_user, b_in_user,
           conv0_W, conv0_b, conv1_W, conv1_b,
           mlp_W1, mlp_b1, mlp_W2, mlp_b2, mlp_W3, mlp_b3,
           W_out_item, b_out_item, W_out_user, b_out_user):
    f32 = jnp.float32

    x_cat = jnp.concatenate([x_item, x_user], axis=0)
    x_pad = jnp.pad(x_cat, ((0, NP - N), (0, 0)))

    pad_idx = jnp.full((E_PAD - E,), N, dtype=jnp.int32)
    src = jnp.concatenate([edge_index[0], pad_idx])
    dst = jnp.concatenate([edge_index[1], pad_idx])
    src3 = src.reshape(NW, CHUNKS, K)
    dst3 = dst.reshape(NW, CHUNKS, K)

    w_in = jnp.stack([W_in_item, W_in_user])
    b_in = jnp.stack([b_in_item, b_in_user]).reshape(2, 1, D)
    w_out = jnp.stack([W_out_item, W_out_user])
    b_out = jnp.stack([b_out_item, b_out_user]).reshape(2, 1, D)
    zeros_tile = jnp.zeros((RPT, D), f32)

    deg_partials = _degrees_sc(src, dst)
    norms = _norms(deg_partials.reshape(2, NW, NP))

    x_proj = _proj(x_pad, w_in, b_in)

    h0 = _conv_in(x_proj, conv0_W, norms)
    agg0 = _edge_agg_sc(h0, src3, dst3, zeros_tile)

    h1 = _conv_mid(agg0, norms, conv0_b.reshape(1, D), conv1_W)
    agg1 = _edge_agg_sc(h1, src3, dst3, zeros_tile)

    out = _decoder(agg1, norms, conv1_b.reshape(1, D),
                   mlp_W1, mlp_b1.reshape(1, 3 * D),
                   mlp_W2, mlp_b2.reshape(1, 2 * D),
                   mlp_W3, mlp_b3.reshape(1, D),
                   w_out, b_out)

    return (out[:N_ITEM], out[N_ITEM:N])


# speedup vs baseline: 1.6496x; 1.0349x over previous
import functools

import jax
import jax.numpy as jnp
from jax import lax
from jax.experimental import pallas as pl
from jax.experimental.pallas import tpu as pltpu
from jax.experimental.pallas import tpu_sc as plsc

N_ITEM = 6000
N_USER = 4000
N = N_ITEM + N_USER
E = 320000
D = 128

NC = 2
NS = 16
NW = NC * NS

RB = 400
NP = 10112
NBLK = -(-NP // RB)
ITEM_BLKS = N_ITEM // RB
HR = NP // 128

K = 128
EPT = 10240
E_PAD = NW * EPT
CHUNKS = EPT // K
NSLOT = 2
CH0 = 114
CH1 = 2 * CHUNKS - CH0
RPT = NP // NS

_mesh = plsc.VectorSubcoreMesh(core_axis_name="c", subcore_axis_name="s")


@functools.partial(
    pl.kernel,
    out_type=jax.ShapeDtypeStruct((2, NW, HR, 128), jnp.float32),
    mesh=_mesh,
    scratch_types=[
        pltpu.VMEM((1024,), jnp.int32),
        pltpu.VMEM((HR, 128), jnp.float32),
        pltpu.VMEM((HR, 128), jnp.float32),
    ],
    compiler_params=pltpu.CompilerParams(needs_layout_passes=False),
)
def _degrees_sc(src_hbm, dst_hbm, out_hbm, ebuf, hs, hd):
    c = lax.axis_index("c")
    s = lax.axis_index("s")
    wid = c * NS + s
    base = wid * EPT

    zero16 = jnp.zeros((16,), jnp.float32)
    one16 = jnp.ones((16,), jnp.float32)

    def zbody(j, _):
        hs[j // 8, pl.ds((j % 8) * 16, 16)] = zero16
        hd[j // 8, pl.ds((j % 8) * 16, 16)] = zero16
        return _

    lax.fori_loop(0, HR * 8, zbody, 0)

    def run(idx_hbm, hist):
        def cbody(ci, _):
            pltpu.sync_copy(idx_hbm.at[pl.ds(base + ci * 1024, 1024)], ebuf)

            def ibody(j, _):
                idx = ebuf[pl.ds(j * 16, 16)]
                plsc.addupdate_scatter(
                    hist,
                    [lax.shift_right_logical(idx, 7),
                     lax.bitwise_and(idx, 127)],
                    one16,
                )
                return _

            lax.fori_loop(0, 64, ibody, 0)
            return _

        lax.fori_loop(0, EPT // 1024, cbody, 0)

    run(src_hbm, hs)
    run(dst_hbm, hd)
    pltpu.sync_copy(hs, out_hbm.at[0, wid])
    pltpu.sync_copy(hd, out_hbm.at[1, wid])


@functools.partial(
    pl.kernel,
    out_type=jax.ShapeDtypeStruct((NC, NP, D), jnp.float32),
    mesh=_mesh,
    scratch_types=(
        [pltpu.VMEM((K,), jnp.int32) for _ in range(2 * NSLOT)]
        + [pltpu.VMEM((K, D), jnp.float32) for _ in range(NSLOT)]
        + [pltpu.VMEM_SHARED((NP, D), jnp.float32)]
        + [pltpu.SemaphoreType.DMA for _ in range(NSLOT)]
    ),
    compiler_params=pltpu.CompilerParams(needs_layout_passes=False),
)
def _edge_agg_sc(h_hbm, src_hbm, dst_hbm, zeros_hbm, out_hbm, *sc):
    sidx = sc[0:NSLOT]
    didx = sc[NSLOT:2 * NSLOT]
    buf = sc[2 * NSLOT:3 * NSLOT]
    accum = sc[3 * NSLOT]
    sem = sc[3 * NSLOT + 1:]
    c = lax.axis_index("c")
    s = lax.axis_index("s")
    r0 = s * RPT

    base = jnp.where(c == 0, s * (CH0 * K), NS * (CH0 * K) + s * (CH1 * K))
    nch = jnp.where(c == 0, CH0, CH1)

    pltpu.sync_copy(zeros_hbm, accum.at[pl.ds(r0, RPT)])
    plsc.subcore_barrier()

    for b in range(NSLOT):
        pltpu.sync_copy(src_hbm.at[pl.ds(base + b * K, K)], sidx[b])
        pltpu.sync_copy(dst_hbm.at[pl.ds(base + b * K, K)], didx[b])
        pltpu.async_copy(h_hbm.at[sidx[b]], buf[b], sem[b])

    def slot(ci, b):
        pltpu.make_async_copy(h_hbm.at[sidx[b]], buf[b], sem[b]).wait()
        pltpu.sync_copy(buf[b], accum.at[didx[b]], add=True)

        @pl.when(ci + NSLOT < nch)
        def _start_next():
            off = base + (ci + NSLOT) * K
            pltpu.sync_copy(src_hbm.at[pl.ds(off, K)], sidx[b])
            pltpu.sync_copy(dst_hbm.at[pl.ds(off, K)], didx[b])
            pltpu.async_copy(h_hbm.at[sidx[b]], buf[b], sem[b])

    def cbody(i, _):
        c0 = i * NSLOT
        for b in range(NSLOT):
            slot(c0 + b, b)
        return _

    lax.fori_loop(0, nch // NSLOT, cbody, 0)
    plsc.subcore_barrier()
    pltpu.sync_copy(accum.at[pl.ds(r0, RPT)], out_hbm.at[c, pl.ds(r0, RPT)])


def _proj_body(x_ref, w_ref, b_ref, o_ref):
    o_ref[...] = (
        jnp.dot(x_ref[...], w_ref[0], preferred_element_type=jnp.float32)
        + b_ref[0, 0]
    )


def _proj(x_pad, w_stack, b_stack):
    return pl.pallas_call(
        _proj_body,
        grid=(NBLK,),
        in_specs=[
            pl.BlockSpec((RB, D), lambda i: (i, 0)),
            pl.BlockSpec((1, D, D), lambda i: (i // ITEM_BLKS, 0, 0)),
            pl.BlockSpec((1, 1, D), lambda i: (i // ITEM_BLKS, 0, 0)),
        ],
        out_specs=pl.BlockSpec((RB, D), lambda i: (i, 0)),
        out_shape=jax.ShapeDtypeStruct((NP, D), jnp.float32),
    )(x_pad, w_stack, b_stack)


def _norms_body(degp_ref, o_ref):
    deg = jnp.sum(degp_ref[...], axis=1)
    o_ref[...] = lax.rsqrt(jnp.maximum(deg, 1.0)).T


def _norms(deg_partials):
    return pl.pallas_call(
        _norms_body,
        out_shape=jax.ShapeDtypeStruct((NP, 2), jnp.float32),
    )(deg_partials)


def _conv_in_body(x_ref, w_ref, nrm_ref, o_ref):
    h = jnp.dot(x_ref[...], w_ref[...], preferred_element_type=jnp.float32)
    o_ref[...] = h * nrm_ref[:, 0:1]


def _conv_in(x, w, norms):
    return pl.pallas_call(
        _conv_in_body,
        grid=(NBLK,),
        in_specs=[
            pl.BlockSpec((RB, D), lambda i: (i, 0)),
            pl.BlockSpec((D, D), lambda i: (0, 0)),
            pl.BlockSpec((RB, 2), lambda i: (i, 0)),
        ],
        out_specs=pl.BlockSpec((RB, D), lambda i: (i, 0)),
        out_shape=jax.ShapeDtypeStruct((NP, D), jnp.float32),
    )(x, w, norms)


def _conv_mid_body(agg_ref, nrm_ref, b_ref, w_ref, o_ref):
    t = (agg_ref[0] + agg_ref[1]) * nrm_ref[:, 1:2] + b_ref[...]
    h = jnp.dot(t, w_ref[...], preferred_element_type=jnp.float32)
    o_ref[...] = h * nrm_ref[:, 0:1]


def _conv_mid(agg, norms, b_prev, w_next):
    return pl.pallas_call(
        _conv_mid_body,
        grid=(NBLK,),
        in_specs=[
            pl.BlockSpec((NC, RB, D), lambda i: (0, i, 0)),
            pl.BlockSpec((RB, 2), lambda i: (i, 0)),
            pl.BlockSpec((1, D), lambda i: (0, 0)),
            pl.BlockSpec((D, D), lambda i: (0, 0)),
        ],
        out_specs=pl.BlockSpec((RB, D), lambda i: (i, 0)),
        out_shape=jax.ShapeDtypeStruct((NP, D), jnp.float32),
    )(agg, norms, b_prev, w_next)


def _dec_body(agg_ref, nrm_ref, bc_ref, w1_ref, b1_ref, w2_ref, b2_ref,
              w3_ref, b3_ref, wo_ref, bo_ref, o_ref):
    t = (agg_ref[0] + agg_ref[1]) * nrm_ref[:, 1:2] + bc_ref[...]
    d = jax.nn.relu(jnp.dot(t, w1_ref[...],
                            preferred_element_type=jnp.float32) + b1_ref[...])
    d = jax.nn.relu(jnp.dot(d, w2_ref[...],
                            preferred_element_type=jnp.float32) + b2_ref[...])
    d = jax.nn.relu(jnp.dot(d, w3_ref[...],
                            preferred_element_type=jnp.float32) + b3_ref[...])
    o = jnp.dot(d, wo_ref[0], preferred_element_type=jnp.float32) + bo_ref[0, 0]
    o_ref[...] = jax.nn.sigmoid(o)


def _decoder(agg, norms, conv_b, w1, b1, w2, b2, w3, b3, wo_stack, bo_stack):
    return pl.pallas_call(
        _dec_body,
        grid=(NBLK,),
        in_specs=[
            pl.BlockSpec((NC, RB, D), lambda i: (0, i, 0)),
            pl.BlockSpec((RB, 2), lambda i: (i, 0)),
            pl.BlockSpec((1, D), lambda i: (0, 0)),
            pl.BlockSpec((D, 3 * D), lambda i: (0, 0)),
            pl.BlockSpec((1, 3 * D), lambda i: (0, 0)),
            pl.BlockSpec((3 * D, 2 * D), lambda i: (0, 0)),
            pl.BlockSpec((1, 2 * D), lambda i: (0, 0)),
            pl.BlockSpec((2 * D, D), lambda i: (0, 0)),
            pl.BlockSpec((1, D), lambda i: (0, 0)),
            pl.BlockSpec((1, D, D), lambda i: (i // ITEM_BLKS, 0, 0)),
            pl.BlockSpec((1, 1, D), lambda i: (i // ITEM_BLKS, 0, 0)),
        ],
        out_specs=pl.BlockSpec((RB, D), lambda i: (i, 0)),
        out_shape=jax.ShapeDtypeStruct((NP, D), jnp.float32),
    )(agg, norms, conv_b, w1, b1, w2, b2, w3, b3, wo_stack, bo_stack)


def kernel(x_item, x_user, edge_index,
           W_in_item, b_in_item, W_in_user, b_in_user,
           conv0_W, conv0_b, conv1_W, conv1_b,
           mlp_W1, mlp_b1, mlp_W2, mlp_b2, mlp_W3, mlp_b3,
           W_out_item, b_out_item, W_out_user, b_out_user):
    f32 = jnp.float32

    x_cat = jnp.concatenate([x_item, x_user], axis=0)
    x_pad = jnp.pad(x_cat, ((0, NP - N), (0, 0)))

    pad_idx = jnp.full((E_PAD - E,), N, dtype=jnp.int32)
    src = jnp.concatenate([edge_index[0], pad_idx])
    dst = jnp.concatenate([edge_index[1], pad_idx])

    w_in = jnp.stack([W_in_item, W_in_user])
    b_in = jnp.stack([b_in_item, b_in_user]).reshape(2, 1, D)
    w_out = jnp.stack([W_out_item, W_out_user])
    b_out = jnp.stack([b_out_item, b_out_user]).reshape(2, 1, D)
    zeros_tile = jnp.zeros((RPT, D), f32)

    deg_partials = _degrees_sc(src, dst)
    norms = _norms(deg_partials.reshape(2, NW, NP))

    x_proj = _proj(x_pad, w_in, b_in)

    h0 = _conv_in(x_proj, conv0_W, norms)
    agg0 = _edge_agg_sc(h0, src, dst, zeros_tile)

    h1 = _conv_mid(agg0, norms, conv0_b.reshape(1, D), conv1_W)
    agg1 = _edge_agg_sc(h1, src, dst, zeros_tile)

    out = _decoder(agg1, norms, conv1_b.reshape(1, D),
                   mlp_W1, mlp_b1.reshape(1, 3 * D),
                   mlp_W2, mlp_b2.reshape(1, 2 * D),
                   mlp_W3, mlp_b3.reshape(1, D),
                   w_out, b_out)

    return (out[:N_ITEM], out[N_ITEM:N])
